# baseline (speedup 1.0000x reference)
import functools

import jax
import jax.numpy as jnp
from jax.experimental import pallas as pl
from jax.experimental.pallas import tpu as pltpu

_N = 1_000_000
_PAD = 1_048_576
_ROWS = 8192
_BETA = 10.0
_TINY = 1.1754943508222875e-38

_KS0 = 0
_KS1 = 42
_KS2 = 0x1BD11BDA ^ _KS0 ^ _KS1
_ROT_A = (13, 15, 26, 6)
_ROT_B = (17, 29, 16, 24)
_INJECT = ((_KS1, _KS2, 1), (_KS2, _KS0, 2), (_KS0, _KS1, 3),
           (_KS1, _KS2, 4), (_KS2, _KS0, 5))


def _rotl(x, r):
    return (x << jnp.uint32(r)) | (x >> jnp.uint32(32 - r))


def _threefry_bits(idx_u32):
    x1 = jnp.full_like(idx_u32, jnp.uint32(_KS0))
    x2 = idx_u32 + jnp.uint32(_KS1)
    rots = (_ROT_A, _ROT_B, _ROT_A, _ROT_B, _ROT_A)
    for g in range(5):
        for r in rots[g]:
            x1 = x1 + x2
            x2 = _rotl(x2, r) ^ x1
        a, b, c = _INJECT[g]
        x1 = x1 + jnp.uint32(a)
        x2 = x2 + jnp.uint32(b) + jnp.uint32(c)
    return x1 ^ x2


def _gumbel(idx_u32):
    bits = _threefry_bits(idx_u32)
    fb = jax.lax.bitcast_convert_type(
        (bits >> jnp.uint32(9)) | jnp.uint32(0x3F800000), jnp.float32)
    fb = fb - jnp.float32(1.0)
    tiny = jnp.float32(_TINY)
    u = jnp.maximum(tiny, fb + tiny)
    return -jnp.log(-jnp.log(u))


def _dense_body(s_ref, out_ref):
    s = s_ref[...]
    row = jax.lax.broadcasted_iota(jnp.uint32, (_ROWS, 128), 0)
    col = jax.lax.broadcasted_iota(jnp.uint32, (_ROWS, 128), 1)
    idx = row * jnp.uint32(128) + col
    v = _BETA * s + _gumbel(idx)
    m = jnp.max(v)
    big = jnp.int32(0x7FFFFFFF)
    cand = jnp.where(v == m, idx.astype(jnp.int32), big)
    out_ref[0, 0] = jnp.min(cand)


@jax.jit
def _sample(scores):
    pad = jnp.full((_PAD - _N,), -jnp.inf, dtype=jnp.float32)
    s2 = jnp.concatenate([scores, pad]).reshape(_ROWS, 128)
    out = pl.pallas_call(
        _dense_body,
        out_shape=jax.ShapeDtypeStruct((1, 1), jnp.int32),
        out_specs=pl.BlockSpec(memory_space=pltpu.SMEM),
    )(s2)
    return out[0, 0]


def kernel(scores):
    return _sample(scores)
